# Initial kernel scaffold; baseline (speedup 1.0000x reference)
import jax
import jax.numpy as jnp
from jax.experimental import pallas as pl


def kernel(input, weight, k, sorted_expert_indices, sorted_scattered_indices, expert_offsets, gates):
    raise NotImplementedError("write your pallas kernel here")


# speedup vs baseline: 1.0923x; 1.0923x over previous
import functools

import jax
import jax.numpy as jnp
from jax import lax
from jax.experimental import pallas as pl
from jax.experimental.pallas import tpu as pltpu
from jax.experimental.pallas import tpu_sc as plsc

_NC = 2
_NS = 16
_NW = _NC * _NS
_L = 16
_BLK = 256
_EPAD = 16


def _make_dispatch(N, DIN, Nk, E, P, K):
    PP = P // _NW
    n_chunk = PP // _L
    GB = 64
    n_g = PP // GB
    mesh = plsc.VectorSubcoreMesh(
        core_axis_name="c", subcore_axis_name="s",
        num_cores=_NC, num_subcores=_NS)

    @functools.partial(
        pl.kernel,
        out_type=jax.ShapeDtypeStruct((P, DIN), jnp.float32),
        mesh=mesh,
        compiler_params=pltpu.CompilerParams(needs_layout_passes=False),
        scratch_types=[
            pltpu.VMEM((Nk,), jnp.int32),
            pltpu.VMEM((_EPAD,), jnp.int32),
            pltpu.VMEM((_EPAD,), jnp.int32),
            pltpu.VMEM((_EPAD,), jnp.int32),
            [pltpu.VMEM((GB,), jnp.int32) for _ in range(n_g)],
            pltpu.VMEM((GB, DIN), jnp.float32),
            pltpu.SemaphoreType.DMA,
        ],
    )
    def dispatch(ssi_hbm, ps_hbm, gs_hbm, ge_hbm, inp_hbm, x_hbm,
                 ssi_v, ps_v, gs_v, ge_v, tok_vs, rows_v, sem):
        wid = lax.axis_index("s") * _NC + lax.axis_index("c")
        base = wid * PP
        pltpu.sync_copy(ssi_hbm, ssi_v)
        pltpu.sync_copy(ps_hbm, ps_v)
        pltpu.sync_copy(gs_hbm, gs_v)
        pltpu.sync_copy(ge_hbm, ge_v)
        ps_all = ps_v[...]
        for j in range(n_chunk):
            p = base + j * _L + lax.iota(jnp.int32, _L)
            e = jnp.zeros((_L,), jnp.int32)
            for ei in range(1, E):
                e = e + (p >= ps_all[ei]).astype(jnp.int32)
            ps_g = plsc.load_gather(ps_v, [e])
            gs_g = plsc.load_gather(gs_v, [e])
            ge_g = plsc.load_gather(ge_v, [e])
            r = p - ps_g + gs_g
            r = jnp.where(r < ge_g, r, 0)
            q = plsc.load_gather(ssi_v, [r])
            tok = q // K
            tok_vs[j // (GB // _L)][pl.ds((j % (GB // _L)) * _L, _L)] = tok
        for c in range(n_g):
            pltpu.async_copy(inp_hbm.at[tok_vs[c]], rows_v, sem).wait()
            pltpu.sync_copy(rows_v, x_hbm.at[pl.ds(base + c * GB, GB), :])

    return dispatch


def _make_matmul(NBP, DIN, DOUT):
    def body(be_ref, x_ref, w_ref, y_ref):
        del be_ref
        y_ref[...] = jnp.dot(x_ref[...], w_ref[0],
                             preferred_element_type=jnp.float32)

    grid_spec = pltpu.PrefetchScalarGridSpec(
        num_scalar_prefetch=1,
        grid=(NBP,),
        in_specs=[
            pl.BlockSpec((_BLK, DIN), lambda b, be: (b, 0)),
            pl.BlockSpec((1, DIN, DOUT), lambda b, be: (be[b], 0, 0)),
        ],
        out_specs=pl.BlockSpec((_BLK, DOUT), lambda b, be: (b, 0)),
    )
    return pl.pallas_call(
        body, grid_spec=grid_spec,
        out_shape=jax.ShapeDtypeStruct((NBP * _BLK, DOUT), jnp.float32))


def _make_combine(N, DOUT, Nk, E, K):
    TPT = N // _NW
    TCK = 32
    n_chunk = TPT // TCK
    QC = TCK * K
    QT = TPT * K
    mesh = plsc.VectorSubcoreMesh(
        core_axis_name="c", subcore_axis_name="s",
        num_cores=_NC, num_subcores=_NS)

    @functools.partial(
        pl.kernel,
        out_type=jax.ShapeDtypeStruct((N, DOUT), jnp.float32),
        mesh=mesh,
        compiler_params=pltpu.CompilerParams(needs_layout_passes=False),
        scratch_types=[
            pltpu.VMEM((Nk,), jnp.int32),
            pltpu.VMEM((_EPAD,), jnp.int32),
            pltpu.VMEM((_EPAD,), jnp.int32),
            pltpu.VMEM((_EPAD,), jnp.int32),
            pltpu.VMEM((QT,), jnp.int32),
            pltpu.VMEM((QC,), jnp.int32),
            pltpu.VMEM((QT,), jnp.float32),
            pltpu.VMEM((QC, DOUT), jnp.float32),
            pltpu.VMEM((TCK, DOUT), jnp.float32),
            pltpu.SemaphoreType.DMA,
        ],
    )
    def combine(ssi_hbm, ps_hbm, gs_hbm, ge_hbm, gates_hbm, y_hbm, out_hbm,
                ssi_v, ps_v, gs_v, ge_v, r_v, idx_v, g_v, y_v, o_v, sem):
        wid = lax.axis_index("s") * _NC + lax.axis_index("c")
        qlo = wid * QT
        pltpu.sync_copy(ssi_hbm, ssi_v)
        pltpu.sync_copy(ps_hbm, ps_v)
        pltpu.sync_copy(gs_hbm, gs_v)
        pltpu.sync_copy(ge_hbm, ge_v)
        pltpu.sync_copy(gates_hbm.at[pl.ds(qlo, QT)], g_v)

        @pl.loop(0, Nk // _L)
        def _(j):
            rr = j * _L + lax.iota(jnp.int32, _L)
            qv = ssi_v[pl.ds(j * _L, _L)]
            lq = qv - qlo
            m = (lq >= 0) & (lq < QT)
            plsc.store_scatter(r_v, [jnp.where(m, lq, 0)], rr, mask=m)

        ge_all = ge_v[...]
        for c in range(n_chunk):
            for jj in range(QC // _L):
                r = r_v[pl.ds(c * QC + jj * _L, _L)]
                e = jnp.zeros((_L,), jnp.int32)
                for ei in range(E - 1):
                    e = e + (r >= ge_all[ei]).astype(jnp.int32)
                ps_g = plsc.load_gather(ps_v, [e])
                gs_g = plsc.load_gather(gs_v, [e])
                idx_v[pl.ds(jj * _L, _L)] = r - gs_g + ps_g
            pltpu.async_copy(y_hbm.at[idx_v], y_v, sem).wait()

            @pl.loop(0, TCK)
            def _(t):
                gb = []
                for s in range(K):
                    gb.append(plsc.load_gather(
                        g_v, [jnp.full((_L,), c * QC + t * K + s, jnp.int32)]))
                for lg in range(DOUT // _L):
                    sl = pl.ds(lg * _L, _L)
                    acc = jnp.zeros((_L,), jnp.float32)
                    for s in range(K):
                        acc = acc + gb[s] * y_v[t * K + s, sl]
                    o_v[t, sl] = acc

            pltpu.sync_copy(
                o_v, out_hbm.at[pl.ds(wid * TPT + c * TCK, TCK), :])

    return combine


def kernel(input, weight, k, sorted_expert_indices, sorted_scattered_indices,
           expert_offsets, gates):
    del sorted_expert_indices, k
    N, DIN = input.shape
    E, _, DOUT = weight.shape
    Nk = sorted_scattered_indices.shape[0]
    K = Nk // N
    NB = Nk // _BLK
    NBP = NB + E
    P = NBP * _BLK

    offs = expert_offsets.astype(jnp.int32)
    gstart = jnp.concatenate([jnp.zeros((1,), jnp.int32), offs[:-1]])
    gend = offs
    sizes = gend - gstart
    padded = ((sizes + _BLK - 1) // _BLK) * _BLK
    nblk = padded // _BLK
    cumblk = jnp.cumsum(nblk).astype(jnp.int32)
    pstart = jnp.concatenate(
        [jnp.zeros((1,), jnp.int32), jnp.cumsum(padded)[:-1].astype(jnp.int32)])
    block_expert = jnp.minimum(
        jnp.searchsorted(cumblk, jnp.arange(NBP, dtype=jnp.int32),
                         side="right"),
        E - 1).astype(jnp.int32)

    def pad16(a):
        return jnp.pad(a, (0, _EPAD - E), mode="edge")

    ssi = sorted_scattered_indices.astype(jnp.int32)
    gates_flat = gates.reshape(-1).astype(jnp.float32)

    x_padded = _make_dispatch(N, DIN, Nk, E, P, K)(
        ssi, pad16(pstart), pad16(gstart), pad16(gend), input)
    y = _make_matmul(NBP, DIN, DOUT)(block_expert, x_padded, weight)
    out = _make_combine(N, DOUT, Nk, E, K)(
        ssi, pad16(pstart), pad16(gstart), pad16(gend), gates_flat, y)
    return out
